# baseline (speedup 1.0000x reference)
import functools

import jax
import jax.numpy as jnp
from jax import lax
from jax.experimental import pallas as pl
from jax.experimental.pallas import tpu as pltpu
from jax.experimental.pallas import tpu_sc as plsc

B = 1024
L = 200
NUM_BUCKETS = 262144
EMBED_DIM = 128
MODEL_DIM = 512
N_TOK = B * L

C1 = 1000003 % NUM_BUCKETS
C2 = (1000003 * 1000003) % NUM_BUCKETS
MASK = NUM_BUCKETS - 1

CHUNK = 128


def _sc_hash_gather(ids32, p1, p2, table):
    n_tok = ids32.shape[0]
    info = plsc.get_sparse_core_info()
    nw = info.num_cores * info.num_subcores
    tpw = n_tok // nw
    n_chunks = tpw // CHUNK
    mesh = plsc.VectorSubcoreMesh(core_axis_name="c", subcore_axis_name="s")

    @functools.partial(
        pl.kernel,
        mesh=mesh,
        out_type=jax.ShapeDtypeStruct((n_tok, EMBED_DIM), jnp.float32),
        scratch_types=[
            pltpu.VMEM((tpw,), jnp.int32),
            pltpu.VMEM((tpw,), jnp.int32),
            pltpu.VMEM((tpw,), jnp.int32),
            pltpu.VMEM((n_chunks, CHUNK), jnp.int32),
            pltpu.VMEM((CHUNK, EMBED_DIM), jnp.float32),
            pltpu.VMEM((CHUNK, EMBED_DIM), jnp.float32),
            pltpu.VMEM((CHUNK, EMBED_DIM), jnp.float32),
            pltpu.VMEM((CHUNK, EMBED_DIM), jnp.float32),
            pltpu.SemaphoreType.DMA,
            pltpu.SemaphoreType.DMA,
            pltpu.SemaphoreType.DMA,
            pltpu.SemaphoreType.DMA,
            pltpu.SemaphoreType.DMA,
            pltpu.SemaphoreType.DMA,
            pltpu.SemaphoreType.DMA,
            pltpu.SemaphoreType.DMA,
        ],
    )
    def k(ids_hbm, p1_hbm, p2_hbm, table_hbm, emb_hbm,
          ids_v, p1_v, p2_v, idx_v, row0_v, row1_v, row2_v, row3_v,
          gs0, gs1, gs2, gs3, os0, os1, os2, os3):
        wid = (lax.axis_index("s").astype(jnp.int32) * jnp.int32(info.num_cores)
               + lax.axis_index("c").astype(jnp.int32))
        base = wid * jnp.int32(tpw)
        pltpu.sync_copy(ids_hbm.at[pl.ds(base, tpw)], ids_v)
        pltpu.sync_copy(p1_hbm.at[pl.ds(base, tpw)], p1_v)
        pltpu.sync_copy(p2_hbm.at[pl.ds(base, tpw)], p2_v)

        c1 = jnp.int32(C1)
        c2 = jnp.int32(C2)
        mask = jnp.int32(MASK)

        @pl.loop(jnp.int32(0), jnp.int32(n_chunks))
        def hash_body(c):
            for j in range(CHUNK // 16):
                off = c * jnp.int32(CHUNK) + jnp.int32(j * 16)
                h = (p2_v[pl.ds(off, 16)] * c2
                     + p1_v[pl.ds(off, 16)] * c1
                     + ids_v[pl.ds(off, 16)]) & mask
                idx_v[c, pl.ds(j * 16, 16)] = h

        rows = (row0_v, row1_v, row2_v, row3_v)
        gsems = (gs0, gs1, gs2, gs3)
        osems = (os0, os1, os2, os3)
        NB = 4
        LEAD = 2

        def gather_start(c, b):
            pltpu.make_async_copy(
                table_hbm.at[idx_v.at[c]], rows[b], gsems[b]).start()

        def gather_wait(c, b):
            pltpu.make_async_copy(
                table_hbm.at[idx_v.at[c]], rows[b], gsems[b]).wait()

        def out_start(c, b):
            pltpu.make_async_copy(
                rows[b],
                emb_hbm.at[pl.ds(base + c * jnp.int32(CHUNK), CHUNK)],
                osems[b]).start()

        def out_wait(c, b):
            pltpu.make_async_copy(
                rows[b],
                emb_hbm.at[pl.ds(base + c * jnp.int32(CHUNK), CHUNK)],
                osems[b]).wait()

        assert (n_chunks - NB - LEAD) % NB == 0 and n_chunks > NB + LEAD

        for b in range(LEAD):
            gather_start(jnp.int32(b), b)
        for c in range(NB):
            b = c % NB
            gather_wait(jnp.int32(c), b)
            out_start(jnp.int32(c), b)
            if c >= LEAD:
                out_wait(jnp.int32(c - LEAD), (c - LEAD) % NB)
            gather_start(jnp.int32(c + LEAD), (c + LEAD) % NB)

        @pl.loop(jnp.int32(1), jnp.int32((n_chunks - LEAD) // NB))
        def gather_body(g):
            for b in range(NB):
                c = g * jnp.int32(NB) + jnp.int32(b)
                gather_wait(c, b)
                out_start(c, b)
                bn = (b + LEAD) % NB
                out_wait(c - jnp.int32(LEAD), bn)
                gather_start(c + jnp.int32(LEAD), bn)

        for c in range(n_chunks - LEAD, n_chunks):
            b = c % NB
            gather_wait(jnp.int32(c), b)
            out_start(jnp.int32(c), b)
            out_wait(jnp.int32(c - LEAD), (c - LEAD) % NB)
        for c in range(n_chunks - LEAD, n_chunks):
            out_wait(jnp.int32(c), c % NB)

    return k(ids32, p1, p2, table)


def _mm_body(a_ref, w_ref, o_ref):
    o_ref[...] = jnp.dot(a_ref[...], w_ref[...],
                         preferred_element_type=jnp.float32)


def _mm_body_acc(a_ref, w_ref, oin_ref, o_ref):
    del oin_ref
    o_ref[...] = jnp.dot(a_ref[...], w_ref[...],
                         preferred_element_type=jnp.float32)


TM = 2048
SLICES = 5


def _tc_matmul_slice(emb, wt, out_buf, block0):
    nb = emb.shape[0] // TM
    if out_buf is None:
        return pl.pallas_call(
            _mm_body,
            grid=(nb,),
            in_specs=[
                pl.BlockSpec((TM, EMBED_DIM), lambda i: (i, i - i)),
                pl.BlockSpec((EMBED_DIM, MODEL_DIM),
                             lambda i: (i - i, i - i)),
            ],
            out_specs=pl.BlockSpec(
                (TM, MODEL_DIM), lambda i: (i + block0, i - i)),
            out_shape=jax.ShapeDtypeStruct((N_TOK, MODEL_DIM), jnp.float32),
        )(emb, wt)
    return pl.pallas_call(
        _mm_body_acc,
        grid=(nb,),
        in_specs=[
            pl.BlockSpec((TM, EMBED_DIM), lambda i: (i, i - i)),
            pl.BlockSpec((EMBED_DIM, MODEL_DIM), lambda i: (i - i, i - i)),
            pl.BlockSpec((TM, MODEL_DIM), lambda i: (i + block0, i - i)),
        ],
        out_specs=pl.BlockSpec(
            (TM, MODEL_DIM), lambda i: (i + block0, i - i)),
        out_shape=jax.ShapeDtypeStruct((N_TOK, MODEL_DIM), jnp.float32),
        input_output_aliases={2: 0},
    )(emb, wt, out_buf)


def kernel(ids, table, W):
    ids32 = ids.astype(jnp.int32)
    p1 = jnp.concatenate([ids32[:, :1], ids32[:, :-1]], axis=1)
    p2 = jnp.concatenate([ids32[:, :2], ids32[:, :-2]], axis=1)
    ids_f = ids32.reshape(-1)
    p1_f = p1.reshape(-1)
    p2_f = p2.reshape(-1)
    wt = W.T

    S = N_TOK // SLICES
    embs = [
        _sc_hash_gather(ids_f[k * S:(k + 1) * S], p1_f[k * S:(k + 1) * S],
                        p2_f[k * S:(k + 1) * S], table)
        for k in range(SLICES)
    ]
    out = None
    for k in range(SLICES):
        out = _tc_matmul_slice(embs[k], wt, out, k * (S // TM))
    return out.reshape(B, L, MODEL_DIM)


# speedup vs baseline: 1.4127x; 1.4127x over previous
import functools

import jax
import jax.numpy as jnp
from jax import lax
from jax.experimental import pallas as pl
from jax.experimental.pallas import tpu as pltpu
from jax.experimental.pallas import tpu_sc as plsc

B = 1024
L = 200
NUM_BUCKETS = 262144
EMBED_DIM = 128
MODEL_DIM = 512
N_TOK = B * L

C1 = 1000003 % NUM_BUCKETS
C2 = (1000003 * 1000003) % NUM_BUCKETS
MASK = NUM_BUCKETS - 1

CHUNK = 128


def _sc_hash_gather(ids32, p1, p2, table):
    n_tok = ids32.shape[0]
    info = plsc.get_sparse_core_info()
    nw = info.num_cores * info.num_subcores
    tpw = n_tok // nw
    n_chunks = tpw // CHUNK
    mesh = plsc.VectorSubcoreMesh(core_axis_name="c", subcore_axis_name="s")

    @functools.partial(
        pl.kernel,
        mesh=mesh,
        out_type=jax.ShapeDtypeStruct((n_tok, EMBED_DIM), jnp.float32),
        scratch_types=[
            pltpu.VMEM((tpw,), jnp.int32),
            pltpu.VMEM((tpw,), jnp.int32),
            pltpu.VMEM((tpw,), jnp.int32),
            pltpu.VMEM((n_chunks, CHUNK), jnp.int32),
            pltpu.VMEM((CHUNK, EMBED_DIM), jnp.float32),
            pltpu.VMEM((CHUNK, EMBED_DIM), jnp.float32),
            pltpu.VMEM((CHUNK, EMBED_DIM), jnp.float32),
            pltpu.VMEM((CHUNK, EMBED_DIM), jnp.float32),
            pltpu.SemaphoreType.DMA,
            pltpu.SemaphoreType.DMA,
            pltpu.SemaphoreType.DMA,
            pltpu.SemaphoreType.DMA,
            pltpu.SemaphoreType.DMA,
            pltpu.SemaphoreType.DMA,
            pltpu.SemaphoreType.DMA,
            pltpu.SemaphoreType.DMA,
        ],
    )
    def k(ids_hbm, p1_hbm, p2_hbm, table_hbm, emb_hbm,
          ids_v, p1_v, p2_v, idx_v, row0_v, row1_v, row2_v, row3_v,
          gs0, gs1, gs2, gs3, os0, os1, os2, os3):
        wid = (lax.axis_index("s").astype(jnp.int32) * jnp.int32(info.num_cores)
               + lax.axis_index("c").astype(jnp.int32))
        base = wid * jnp.int32(tpw)
        pltpu.sync_copy(ids_hbm.at[pl.ds(base, tpw)], ids_v)
        pltpu.sync_copy(p1_hbm.at[pl.ds(base, tpw)], p1_v)
        pltpu.sync_copy(p2_hbm.at[pl.ds(base, tpw)], p2_v)

        c1 = jnp.int32(C1)
        c2 = jnp.int32(C2)
        mask = jnp.int32(MASK)

        @pl.loop(jnp.int32(0), jnp.int32(n_chunks))
        def hash_body(c):
            for j in range(CHUNK // 16):
                off = c * jnp.int32(CHUNK) + jnp.int32(j * 16)
                h = (p2_v[pl.ds(off, 16)] * c2
                     + p1_v[pl.ds(off, 16)] * c1
                     + ids_v[pl.ds(off, 16)]) & mask
                idx_v[c, pl.ds(j * 16, 16)] = h

        rows = (row0_v, row1_v, row2_v, row3_v)
        gsems = (gs0, gs1, gs2, gs3)
        osems = (os0, os1, os2, os3)
        NB = 4
        LEAD = 2

        def gather_start(c, b):
            pltpu.make_async_copy(
                table_hbm.at[idx_v.at[c]], rows[b], gsems[b]).start()

        def gather_wait(c, b):
            pltpu.make_async_copy(
                table_hbm.at[idx_v.at[c]], rows[b], gsems[b]).wait()

        def out_start(c, b):
            pltpu.make_async_copy(
                rows[b],
                emb_hbm.at[pl.ds(base + c * jnp.int32(CHUNK), CHUNK)],
                osems[b]).start()

        def out_wait(c, b):
            pltpu.make_async_copy(
                rows[b],
                emb_hbm.at[pl.ds(base + c * jnp.int32(CHUNK), CHUNK)],
                osems[b]).wait()

        assert (n_chunks - NB - LEAD) % NB == 0 and n_chunks > NB + LEAD

        for b in range(LEAD):
            gather_start(jnp.int32(b), b)
        for c in range(NB):
            b = c % NB
            gather_wait(jnp.int32(c), b)
            out_start(jnp.int32(c), b)
            if c >= LEAD:
                out_wait(jnp.int32(c - LEAD), (c - LEAD) % NB)
            gather_start(jnp.int32(c + LEAD), (c + LEAD) % NB)

        @pl.loop(jnp.int32(1), jnp.int32((n_chunks - LEAD) // NB))
        def gather_body(g):
            for b in range(NB):
                c = g * jnp.int32(NB) + jnp.int32(b)
                gather_wait(c, b)
                out_start(c, b)
                bn = (b + LEAD) % NB
                out_wait(c - jnp.int32(LEAD), bn)
                gather_start(c + jnp.int32(LEAD), bn)

        for c in range(n_chunks - LEAD, n_chunks):
            b = c % NB
            gather_wait(jnp.int32(c), b)
            out_start(jnp.int32(c), b)
            out_wait(jnp.int32(c - LEAD), (c - LEAD) % NB)
        for c in range(n_chunks - LEAD, n_chunks):
            out_wait(jnp.int32(c), c % NB)

    return k(ids32, p1, p2, table)


def _mm_body(a_ref, w_ref, o_ref):
    o_ref[...] = jnp.dot(a_ref[...], w_ref[...],
                         preferred_element_type=jnp.float32)


def _mm_body_acc(a_ref, w_ref, oin_ref, o_ref):
    del oin_ref
    o_ref[...] = jnp.dot(a_ref[...], w_ref[...],
                         preferred_element_type=jnp.float32)


TM = 4096
SLICES = 1


def _tc_matmul_slice(emb, wt, out_buf, block0):
    nb = emb.shape[0] // TM
    if out_buf is None:
        return pl.pallas_call(
            _mm_body,
            grid=(nb,),
            in_specs=[
                pl.BlockSpec((TM, EMBED_DIM), lambda i: (i, i - i)),
                pl.BlockSpec((EMBED_DIM, MODEL_DIM),
                             lambda i: (i - i, i - i)),
            ],
            out_specs=pl.BlockSpec(
                (TM, MODEL_DIM), lambda i: (i + block0, i - i)),
            out_shape=jax.ShapeDtypeStruct((N_TOK, MODEL_DIM), jnp.float32),
        )(emb, wt)
    return pl.pallas_call(
        _mm_body_acc,
        grid=(nb,),
        in_specs=[
            pl.BlockSpec((TM, EMBED_DIM), lambda i: (i, i - i)),
            pl.BlockSpec((EMBED_DIM, MODEL_DIM), lambda i: (i - i, i - i)),
            pl.BlockSpec((TM, MODEL_DIM), lambda i: (i + block0, i - i)),
        ],
        out_specs=pl.BlockSpec(
            (TM, MODEL_DIM), lambda i: (i + block0, i - i)),
        out_shape=jax.ShapeDtypeStruct((N_TOK, MODEL_DIM), jnp.float32),
        input_output_aliases={2: 0},
    )(emb, wt, out_buf)


def kernel(ids, table, W):
    ids32 = ids.astype(jnp.int32)
    p1 = jnp.concatenate([ids32[:, :1], ids32[:, :-1]], axis=1)
    p2 = jnp.concatenate([ids32[:, :2], ids32[:, :-2]], axis=1)
    ids_f = ids32.reshape(-1)
    p1_f = p1.reshape(-1)
    p2_f = p2.reshape(-1)
    wt = W.T

    S = N_TOK // SLICES
    embs = [
        _sc_hash_gather(ids_f[k * S:(k + 1) * S], p1_f[k * S:(k + 1) * S],
                        p2_f[k * S:(k + 1) * S], table)
        for k in range(SLICES)
    ]
    out = None
    for k in range(SLICES):
        out = _tc_matmul_slice(embs[k], wt, out, k * (S // TM))
    return out.reshape(B, L, MODEL_DIM)


# speedup vs baseline: 1.4309x; 1.0129x over previous
import functools

import jax
import jax.numpy as jnp
from jax import lax
from jax.experimental import pallas as pl
from jax.experimental.pallas import tpu as pltpu
from jax.experimental.pallas import tpu_sc as plsc

B = 1024
L = 200
NUM_BUCKETS = 262144
EMBED_DIM = 128
MODEL_DIM = 512
N_TOK = B * L

C1 = 1000003 % NUM_BUCKETS
C2 = (1000003 * 1000003) % NUM_BUCKETS
MASK = NUM_BUCKETS - 1

CHUNK = 128


def _sc_hash_gather(ids32, p1, p2, table):
    n_tok = ids32.shape[0]
    info = plsc.get_sparse_core_info()
    nw = info.num_cores * info.num_subcores
    tpw = n_tok // nw
    n_chunks = tpw // CHUNK
    mesh = plsc.VectorSubcoreMesh(core_axis_name="c", subcore_axis_name="s")

    @functools.partial(
        pl.kernel,
        mesh=mesh,
        out_type=jax.ShapeDtypeStruct((n_tok, EMBED_DIM), jnp.float32),
        scratch_types=[
            pltpu.VMEM((tpw,), jnp.int32),
            pltpu.VMEM((tpw,), jnp.int32),
            pltpu.VMEM((tpw,), jnp.int32),
            pltpu.VMEM((n_chunks, CHUNK), jnp.int32),
            pltpu.VMEM((CHUNK, EMBED_DIM), jnp.float32),
            pltpu.VMEM((CHUNK, EMBED_DIM), jnp.float32),
            pltpu.VMEM((CHUNK, EMBED_DIM), jnp.float32),
            pltpu.VMEM((CHUNK, EMBED_DIM), jnp.float32),
            pltpu.SemaphoreType.DMA,
            pltpu.SemaphoreType.DMA,
            pltpu.SemaphoreType.DMA,
            pltpu.SemaphoreType.DMA,
            pltpu.SemaphoreType.DMA,
            pltpu.SemaphoreType.DMA,
            pltpu.SemaphoreType.DMA,
            pltpu.SemaphoreType.DMA,
        ],
    )
    def k(ids_hbm, p1_hbm, p2_hbm, table_hbm, emb_hbm,
          ids_v, p1_v, p2_v, idx_v, row0_v, row1_v, row2_v, row3_v,
          gs0, gs1, gs2, gs3, os0, os1, os2, os3):
        wid = (lax.axis_index("s").astype(jnp.int32) * jnp.int32(info.num_cores)
               + lax.axis_index("c").astype(jnp.int32))
        base = wid * jnp.int32(tpw)
        pltpu.sync_copy(ids_hbm.at[pl.ds(base, tpw)], ids_v)
        pltpu.sync_copy(p1_hbm.at[pl.ds(base, tpw)], p1_v)
        pltpu.sync_copy(p2_hbm.at[pl.ds(base, tpw)], p2_v)

        c1 = jnp.int32(C1)
        c2 = jnp.int32(C2)
        mask = jnp.int32(MASK)

        @pl.loop(jnp.int32(0), jnp.int32(n_chunks))
        def hash_body(c):
            for j in range(CHUNK // 16):
                off = c * jnp.int32(CHUNK) + jnp.int32(j * 16)
                h = (p2_v[pl.ds(off, 16)] * c2
                     + p1_v[pl.ds(off, 16)] * c1
                     + ids_v[pl.ds(off, 16)]) & mask
                idx_v[c, pl.ds(j * 16, 16)] = h

        rows = (row0_v, row1_v, row2_v, row3_v)
        gsems = (gs0, gs1, gs2, gs3)
        osems = (os0, os1, os2, os3)
        NB = 4
        LEAD = 2

        def gather_start(c, b):
            pltpu.make_async_copy(
                table_hbm.at[idx_v.at[c]], rows[b], gsems[b]).start()

        def gather_wait(c, b):
            pltpu.make_async_copy(
                table_hbm.at[idx_v.at[c]], rows[b], gsems[b]).wait()

        def out_start(c, b):
            pltpu.make_async_copy(
                rows[b],
                emb_hbm.at[pl.ds(base + c * jnp.int32(CHUNK), CHUNK)],
                osems[b]).start()

        def out_wait(c, b):
            pltpu.make_async_copy(
                rows[b],
                emb_hbm.at[pl.ds(base + c * jnp.int32(CHUNK), CHUNK)],
                osems[b]).wait()

        assert (n_chunks - NB - LEAD) % NB == 0 and n_chunks > NB + LEAD

        for b in range(LEAD):
            gather_start(jnp.int32(b), b)
        for c in range(NB):
            b = c % NB
            gather_wait(jnp.int32(c), b)
            out_start(jnp.int32(c), b)
            if c >= LEAD:
                out_wait(jnp.int32(c - LEAD), (c - LEAD) % NB)
            gather_start(jnp.int32(c + LEAD), (c + LEAD) % NB)

        @pl.loop(jnp.int32(1), jnp.int32((n_chunks - LEAD) // NB))
        def gather_body(g):
            for b in range(NB):
                c = g * jnp.int32(NB) + jnp.int32(b)
                gather_wait(c, b)
                out_start(c, b)
                bn = (b + LEAD) % NB
                out_wait(c - jnp.int32(LEAD), bn)
                gather_start(c + jnp.int32(LEAD), bn)

        for c in range(n_chunks - LEAD, n_chunks):
            b = c % NB
            gather_wait(jnp.int32(c), b)
            out_start(jnp.int32(c), b)
            out_wait(jnp.int32(c - LEAD), (c - LEAD) % NB)
        for c in range(n_chunks - LEAD, n_chunks):
            out_wait(jnp.int32(c), c % NB)

    return k(ids32, p1, p2, table)


def _mm_body(a_ref, w_ref, o_ref):
    o_ref[...] = jnp.dot(a_ref[...], w_ref[...],
                         preferred_element_type=jnp.float32)


def _mm_body_acc(a_ref, w_ref, oin_ref, o_ref):
    del oin_ref
    o_ref[...] = jnp.dot(a_ref[...], w_ref[...],
                         preferred_element_type=jnp.float32)


TM = 8192
SLICES = 1


def _tc_matmul_slice(emb, wt, out_buf, block0):
    nb = emb.shape[0] // TM
    if out_buf is None:
        return pl.pallas_call(
            _mm_body,
            grid=(nb,),
            in_specs=[
                pl.BlockSpec((TM, EMBED_DIM), lambda i: (i, i - i)),
                pl.BlockSpec((EMBED_DIM, MODEL_DIM),
                             lambda i: (i - i, i - i)),
            ],
            out_specs=pl.BlockSpec(
                (TM, MODEL_DIM), lambda i: (i + block0, i - i)),
            out_shape=jax.ShapeDtypeStruct((N_TOK, MODEL_DIM), jnp.float32),
        )(emb, wt)
    return pl.pallas_call(
        _mm_body_acc,
        grid=(nb,),
        in_specs=[
            pl.BlockSpec((TM, EMBED_DIM), lambda i: (i, i - i)),
            pl.BlockSpec((EMBED_DIM, MODEL_DIM), lambda i: (i - i, i - i)),
            pl.BlockSpec((TM, MODEL_DIM), lambda i: (i + block0, i - i)),
        ],
        out_specs=pl.BlockSpec(
            (TM, MODEL_DIM), lambda i: (i + block0, i - i)),
        out_shape=jax.ShapeDtypeStruct((N_TOK, MODEL_DIM), jnp.float32),
        input_output_aliases={2: 0},
    )(emb, wt, out_buf)


def kernel(ids, table, W):
    ids32 = ids.astype(jnp.int32)
    p1 = jnp.concatenate([ids32[:, :1], ids32[:, :-1]], axis=1)
    p2 = jnp.concatenate([ids32[:, :2], ids32[:, :-2]], axis=1)
    ids_f = ids32.reshape(-1)
    p1_f = p1.reshape(-1)
    p2_f = p2.reshape(-1)
    wt = W.T

    S = N_TOK // SLICES
    embs = [
        _sc_hash_gather(ids_f[k * S:(k + 1) * S], p1_f[k * S:(k + 1) * S],
                        p2_f[k * S:(k + 1) * S], table)
        for k in range(SLICES)
    ]
    out = None
    for k in range(SLICES):
        out = _tc_matmul_slice(embs[k], wt, out, k * (S // TM))
    return out.reshape(B, L, MODEL_DIM)


# speedup vs baseline: 1.4323x; 1.0010x over previous
import functools

import jax
import jax.numpy as jnp
from jax import lax
from jax.experimental import pallas as pl
from jax.experimental.pallas import tpu as pltpu
from jax.experimental.pallas import tpu_sc as plsc

B = 1024
L = 200
NUM_BUCKETS = 262144
EMBED_DIM = 128
MODEL_DIM = 512
N_TOK = B * L

C1 = 1000003 % NUM_BUCKETS
C2 = (1000003 * 1000003) % NUM_BUCKETS
MASK = NUM_BUCKETS - 1

CHUNK = 128


def _sc_hash_gather(ids32, p1, p2, table):
    n_tok = ids32.shape[0]
    info = plsc.get_sparse_core_info()
    nw = info.num_cores * info.num_subcores
    tpw = n_tok // nw
    n_chunks = tpw // CHUNK
    mesh = plsc.VectorSubcoreMesh(core_axis_name="c", subcore_axis_name="s")

    @functools.partial(
        pl.kernel,
        mesh=mesh,
        out_type=jax.ShapeDtypeStruct((n_tok, EMBED_DIM), jnp.float32),
        scratch_types=[
            pltpu.VMEM((tpw,), jnp.int32),
            pltpu.VMEM((tpw,), jnp.int32),
            pltpu.VMEM((tpw,), jnp.int32),
            pltpu.VMEM((n_chunks, CHUNK), jnp.int32),
            pltpu.VMEM((CHUNK, EMBED_DIM), jnp.float32),
            pltpu.VMEM((CHUNK, EMBED_DIM), jnp.float32),
            pltpu.VMEM((CHUNK, EMBED_DIM), jnp.float32),
            pltpu.VMEM((CHUNK, EMBED_DIM), jnp.float32),
            pltpu.SemaphoreType.DMA,
            pltpu.SemaphoreType.DMA,
            pltpu.SemaphoreType.DMA,
            pltpu.SemaphoreType.DMA,
            pltpu.SemaphoreType.DMA,
            pltpu.SemaphoreType.DMA,
            pltpu.SemaphoreType.DMA,
            pltpu.SemaphoreType.DMA,
        ],
    )
    def k(ids_hbm, p1_hbm, p2_hbm, table_hbm, emb_hbm,
          ids_v, p1_v, p2_v, idx_v, row0_v, row1_v, row2_v, row3_v,
          gs0, gs1, gs2, gs3, os0, os1, os2, os3):
        wid = (lax.axis_index("s").astype(jnp.int32) * jnp.int32(info.num_cores)
               + lax.axis_index("c").astype(jnp.int32))
        base = wid * jnp.int32(tpw)
        pltpu.sync_copy(ids_hbm.at[pl.ds(base, tpw)], ids_v)
        pltpu.sync_copy(p1_hbm.at[pl.ds(base, tpw)], p1_v)
        pltpu.sync_copy(p2_hbm.at[pl.ds(base, tpw)], p2_v)

        c1 = jnp.int32(C1)
        c2 = jnp.int32(C2)
        mask = jnp.int32(MASK)

        @pl.loop(jnp.int32(0), jnp.int32(n_chunks))
        def hash_body(c):
            for j in range(CHUNK // 16):
                off = c * jnp.int32(CHUNK) + jnp.int32(j * 16)
                h = (p2_v[pl.ds(off, 16)] * c2
                     + p1_v[pl.ds(off, 16)] * c1
                     + ids_v[pl.ds(off, 16)]) & mask
                idx_v[c, pl.ds(j * 16, 16)] = h

        rows = (row0_v, row1_v, row2_v, row3_v)
        gsems = (gs0, gs1, gs2, gs3)
        osems = (os0, os1, os2, os3)
        NB = 4
        LEAD = 2

        def gather_start(c, b):
            pltpu.make_async_copy(
                table_hbm.at[idx_v.at[c]], rows[b], gsems[b]).start()

        def gather_wait(c, b):
            pltpu.make_async_copy(
                table_hbm.at[idx_v.at[c]], rows[b], gsems[b]).wait()

        def out_start(c, b):
            pltpu.make_async_copy(
                rows[b],
                emb_hbm.at[pl.ds(base + c * jnp.int32(CHUNK), CHUNK)],
                osems[b]).start()

        def out_wait(c, b):
            pltpu.make_async_copy(
                rows[b],
                emb_hbm.at[pl.ds(base + c * jnp.int32(CHUNK), CHUNK)],
                osems[b]).wait()

        assert (n_chunks - NB - LEAD) % NB == 0 and n_chunks > NB + LEAD

        for b in range(LEAD):
            gather_start(jnp.int32(b), b)
        for c in range(NB):
            b = c % NB
            gather_wait(jnp.int32(c), b)
            out_start(jnp.int32(c), b)
            if c >= LEAD:
                out_wait(jnp.int32(c - LEAD), (c - LEAD) % NB)
            gather_start(jnp.int32(c + LEAD), (c + LEAD) % NB)

        @pl.loop(jnp.int32(1), jnp.int32((n_chunks - LEAD) // NB))
        def gather_body(g):
            for b in range(NB):
                c = g * jnp.int32(NB) + jnp.int32(b)
                gather_wait(c, b)
                out_start(c, b)
                bn = (b + LEAD) % NB
                out_wait(c - jnp.int32(LEAD), bn)
                gather_start(c + jnp.int32(LEAD), bn)

        for c in range(n_chunks - LEAD, n_chunks):
            b = c % NB
            gather_wait(jnp.int32(c), b)
            out_start(jnp.int32(c), b)
            out_wait(jnp.int32(c - LEAD), (c - LEAD) % NB)
        for c in range(n_chunks - LEAD, n_chunks):
            out_wait(jnp.int32(c), c % NB)

    return k(ids32, p1, p2, table)


def _mm_body(a_ref, w_ref, o_ref):
    o_ref[...] = jnp.dot(a_ref[...].astype(jnp.bfloat16),
                         w_ref[...].astype(jnp.bfloat16),
                         preferred_element_type=jnp.float32)


def _mm_body_acc(a_ref, w_ref, oin_ref, o_ref):
    del oin_ref
    o_ref[...] = jnp.dot(a_ref[...], w_ref[...],
                         preferred_element_type=jnp.float32)


TM = 8192
SLICES = 1


def _tc_matmul_slice(emb, wt, out_buf, block0):
    nb = emb.shape[0] // TM
    if out_buf is None:
        return pl.pallas_call(
            _mm_body,
            grid=(nb,),
            in_specs=[
                pl.BlockSpec((TM, EMBED_DIM), lambda i: (i, i - i)),
                pl.BlockSpec((EMBED_DIM, MODEL_DIM),
                             lambda i: (i - i, i - i)),
            ],
            out_specs=pl.BlockSpec(
                (TM, MODEL_DIM), lambda i: (i + block0, i - i)),
            out_shape=jax.ShapeDtypeStruct((N_TOK, MODEL_DIM), jnp.float32),
        )(emb, wt)
    return pl.pallas_call(
        _mm_body_acc,
        grid=(nb,),
        in_specs=[
            pl.BlockSpec((TM, EMBED_DIM), lambda i: (i, i - i)),
            pl.BlockSpec((EMBED_DIM, MODEL_DIM), lambda i: (i - i, i - i)),
            pl.BlockSpec((TM, MODEL_DIM), lambda i: (i + block0, i - i)),
        ],
        out_specs=pl.BlockSpec(
            (TM, MODEL_DIM), lambda i: (i + block0, i - i)),
        out_shape=jax.ShapeDtypeStruct((N_TOK, MODEL_DIM), jnp.float32),
        input_output_aliases={2: 0},
    )(emb, wt, out_buf)


def kernel(ids, table, W):
    ids32 = ids.astype(jnp.int32)
    p1 = jnp.concatenate([ids32[:, :1], ids32[:, :-1]], axis=1)
    p2 = jnp.concatenate([ids32[:, :2], ids32[:, :-2]], axis=1)
    ids_f = ids32.reshape(-1)
    p1_f = p1.reshape(-1)
    p2_f = p2.reshape(-1)
    wt = W.T

    S = N_TOK // SLICES
    embs = [
        _sc_hash_gather(ids_f[k * S:(k + 1) * S], p1_f[k * S:(k + 1) * S],
                        p2_f[k * S:(k + 1) * S], table)
        for k in range(SLICES)
    ]
    out = None
    for k in range(SLICES):
        out = _tc_matmul_slice(embs[k], wt, out, k * (S // TM))
    return out.reshape(B, L, MODEL_DIM)


# speedup vs baseline: 1.4332x; 1.0007x over previous
import functools

import jax
import jax.numpy as jnp
from jax import lax
from jax.experimental import pallas as pl
from jax.experimental.pallas import tpu as pltpu
from jax.experimental.pallas import tpu_sc as plsc

B = 1024
L = 200
NUM_BUCKETS = 262144
EMBED_DIM = 128
MODEL_DIM = 512
N_TOK = B * L

C1 = 1000003 % NUM_BUCKETS
C2 = (1000003 * 1000003) % NUM_BUCKETS
MASK = NUM_BUCKETS - 1

CHUNK = 128
_NB = 6
_LEAD = 3


def _sc_hash_gather(ids32, p1, p2, table):
    n_tok = ids32.shape[0]
    info = plsc.get_sparse_core_info()
    nw = info.num_cores * info.num_subcores
    tpw = n_tok // nw
    n_chunks = tpw // CHUNK
    mesh = plsc.VectorSubcoreMesh(core_axis_name="c", subcore_axis_name="s")

    @functools.partial(
        pl.kernel,
        mesh=mesh,
        out_type=jax.ShapeDtypeStruct((n_tok, EMBED_DIM), jnp.float32),
        scratch_types=[
            pltpu.VMEM((tpw,), jnp.int32),
            pltpu.VMEM((tpw,), jnp.int32),
            pltpu.VMEM((tpw,), jnp.int32),
            pltpu.VMEM((n_chunks, CHUNK), jnp.int32),
        ] + [pltpu.VMEM((CHUNK, EMBED_DIM), jnp.float32)] * _NB
          + [pltpu.SemaphoreType.DMA] * (2 * _NB),
    )
    def k(ids_hbm, p1_hbm, p2_hbm, table_hbm, emb_hbm,
          ids_v, p1_v, p2_v, idx_v, *bufs):
        wid = (lax.axis_index("s").astype(jnp.int32) * jnp.int32(info.num_cores)
               + lax.axis_index("c").astype(jnp.int32))
        base = wid * jnp.int32(tpw)
        pltpu.sync_copy(ids_hbm.at[pl.ds(base, tpw)], ids_v)
        pltpu.sync_copy(p1_hbm.at[pl.ds(base, tpw)], p1_v)
        pltpu.sync_copy(p2_hbm.at[pl.ds(base, tpw)], p2_v)

        c1 = jnp.int32(C1)
        c2 = jnp.int32(C2)
        mask = jnp.int32(MASK)

        @pl.loop(jnp.int32(0), jnp.int32(n_chunks))
        def hash_body(c):
            for j in range(CHUNK // 16):
                off = c * jnp.int32(CHUNK) + jnp.int32(j * 16)
                h = (p2_v[pl.ds(off, 16)] * c2
                     + p1_v[pl.ds(off, 16)] * c1
                     + ids_v[pl.ds(off, 16)]) & mask
                idx_v[c, pl.ds(j * 16, 16)] = h

        rows = bufs[:_NB]
        gsems = bufs[_NB:2 * _NB]
        osems = bufs[2 * _NB:]
        NB = _NB
        LEAD = _LEAD

        def gather_start(c, b):
            pltpu.make_async_copy(
                table_hbm.at[idx_v.at[c]], rows[b], gsems[b]).start()

        def gather_wait(c, b):
            pltpu.make_async_copy(
                table_hbm.at[idx_v.at[c]], rows[b], gsems[b]).wait()

        def out_start(c, b):
            pltpu.make_async_copy(
                rows[b],
                emb_hbm.at[pl.ds(base + c * jnp.int32(CHUNK), CHUNK)],
                osems[b]).start()

        def out_wait(c, b):
            pltpu.make_async_copy(
                rows[b],
                emb_hbm.at[pl.ds(base + c * jnp.int32(CHUNK), CHUNK)],
                osems[b]).wait()

        assert n_chunks > NB + LEAD and NB >= 2 * LEAD

        def step_static(c):
            ci = jnp.int32(c)
            gather_wait(ci, c % NB)
            out_start(ci, c % NB)
            if c >= LEAD:
                out_wait(jnp.int32(c - LEAD), (c - LEAD) % NB)
            if c + LEAD < n_chunks:
                gather_start(jnp.int32(c + LEAD), (c + LEAD) % NB)

        n_main = ((n_chunks - LEAD) - NB) // NB
        tail0 = NB + n_main * NB

        for b in range(LEAD):
            gather_start(jnp.int32(b), b)
        for c in range(NB):
            step_static(c)

        @pl.loop(jnp.int32(1), jnp.int32(1 + n_main))
        def gather_body(g):
            for b in range(NB):
                c = g * jnp.int32(NB) + jnp.int32(b)
                gather_wait(c, b)
                out_start(c, b)
                out_wait(c - jnp.int32(LEAD), (b - LEAD) % NB)
                gather_start(c + jnp.int32(LEAD), (b + LEAD) % NB)

        for c in range(tail0, n_chunks):
            step_static(c)
        for c in range(n_chunks - LEAD, n_chunks):
            out_wait(jnp.int32(c), c % NB)

    return k(ids32, p1, p2, table)


def _mm_body(a_ref, w_ref, o_ref):
    o_ref[...] = jnp.dot(a_ref[...], w_ref[...],
                         preferred_element_type=jnp.float32)


def _mm_body_acc(a_ref, w_ref, oin_ref, o_ref):
    del oin_ref
    o_ref[...] = jnp.dot(a_ref[...], w_ref[...],
                         preferred_element_type=jnp.float32)


TM = 8192
SLICES = 1


def _tc_matmul_slice(emb, wt, out_buf, block0):
    nb = emb.shape[0] // TM
    if out_buf is None:
        return pl.pallas_call(
            _mm_body,
            grid=(nb,),
            in_specs=[
                pl.BlockSpec((TM, EMBED_DIM), lambda i: (i, i - i)),
                pl.BlockSpec((EMBED_DIM, MODEL_DIM),
                             lambda i: (i - i, i - i)),
            ],
            out_specs=pl.BlockSpec(
                (TM, MODEL_DIM), lambda i: (i + block0, i - i)),
            out_shape=jax.ShapeDtypeStruct((N_TOK, MODEL_DIM), jnp.float32),
        )(emb, wt)
    return pl.pallas_call(
        _mm_body_acc,
        grid=(nb,),
        in_specs=[
            pl.BlockSpec((TM, EMBED_DIM), lambda i: (i, i - i)),
            pl.BlockSpec((EMBED_DIM, MODEL_DIM), lambda i: (i - i, i - i)),
            pl.BlockSpec((TM, MODEL_DIM), lambda i: (i + block0, i - i)),
        ],
        out_specs=pl.BlockSpec(
            (TM, MODEL_DIM), lambda i: (i + block0, i - i)),
        out_shape=jax.ShapeDtypeStruct((N_TOK, MODEL_DIM), jnp.float32),
        input_output_aliases={2: 0},
    )(emb, wt, out_buf)


def kernel(ids, table, W):
    ids32 = ids.astype(jnp.int32)
    p1 = jnp.concatenate([ids32[:, :1], ids32[:, :-1]], axis=1)
    p2 = jnp.concatenate([ids32[:, :2], ids32[:, :-2]], axis=1)
    ids_f = ids32.reshape(-1)
    p1_f = p1.reshape(-1)
    p2_f = p2.reshape(-1)
    wt = W.T

    S = N_TOK // SLICES
    embs = [
        _sc_hash_gather(ids_f[k * S:(k + 1) * S], p1_f[k * S:(k + 1) * S],
                        p2_f[k * S:(k + 1) * S], table)
        for k in range(SLICES)
    ]
    out = None
    for k in range(SLICES):
        out = _tc_matmul_slice(embs[k], wt, out, k * (S // TM))
    return out.reshape(B, L, MODEL_DIM)


# speedup vs baseline: 1.4391x; 1.0041x over previous
import functools

import jax
import jax.numpy as jnp
from jax import lax
from jax.experimental import pallas as pl
from jax.experimental.pallas import tpu as pltpu
from jax.experimental.pallas import tpu_sc as plsc

B = 1024
L = 200
NUM_BUCKETS = 262144
EMBED_DIM = 128
MODEL_DIM = 512
N_TOK = B * L

C1 = 1000003 % NUM_BUCKETS
C2 = (1000003 * 1000003) % NUM_BUCKETS
MASK = NUM_BUCKETS - 1

CHUNK = 128
_NB = 6
_LEAD = 3


def _sc_hash_gather(ids32, p1, p2, table):
    n_tok = ids32.shape[0]
    info = plsc.get_sparse_core_info()
    nw = info.num_cores * info.num_subcores
    tpw = n_tok // nw
    n_chunks = tpw // CHUNK
    mesh = plsc.VectorSubcoreMesh(core_axis_name="c", subcore_axis_name="s")

    @functools.partial(
        pl.kernel,
        mesh=mesh,
        out_type=jax.ShapeDtypeStruct((n_tok, EMBED_DIM), jnp.float32),
        scratch_types=[
            pltpu.VMEM((tpw,), jnp.int32),
            pltpu.VMEM((tpw,), jnp.int32),
            pltpu.VMEM((tpw,), jnp.int32),
            pltpu.VMEM((n_chunks, CHUNK), jnp.int32),
        ] + [pltpu.VMEM((CHUNK, EMBED_DIM), jnp.float32)] * _NB
          + [pltpu.SemaphoreType.DMA] * (2 * _NB),
    )
    def k(ids_hbm, p1_hbm, p2_hbm, table_hbm, emb_hbm,
          ids_v, p1_v, p2_v, idx_v, *bufs):
        wid = (lax.axis_index("s").astype(jnp.int32) * jnp.int32(info.num_cores)
               + lax.axis_index("c").astype(jnp.int32))
        base = wid * jnp.int32(tpw)
        pltpu.sync_copy(ids_hbm.at[pl.ds(base, tpw)], ids_v)
        pltpu.sync_copy(p1_hbm.at[pl.ds(base, tpw)], p1_v)
        pltpu.sync_copy(p2_hbm.at[pl.ds(base, tpw)], p2_v)

        c1 = jnp.int32(C1)
        c2 = jnp.int32(C2)
        mask = jnp.int32(MASK)

        @pl.loop(jnp.int32(0), jnp.int32(n_chunks))
        def hash_body(c):
            for j in range(CHUNK // 16):
                off = c * jnp.int32(CHUNK) + jnp.int32(j * 16)
                h = (p2_v[pl.ds(off, 16)] * c2
                     + p1_v[pl.ds(off, 16)] * c1
                     + ids_v[pl.ds(off, 16)]) & mask
                idx_v[c, pl.ds(j * 16, 16)] = h

        rows = bufs[:_NB]
        gsems = bufs[_NB:2 * _NB]
        osems = bufs[2 * _NB:]
        NB = _NB
        LEAD = _LEAD

        def gather_start(c, b):
            pltpu.make_async_copy(
                table_hbm.at[idx_v.at[c]], rows[b], gsems[b]).start()

        def gather_wait(c, b):
            pltpu.make_async_copy(
                table_hbm.at[idx_v.at[c]], rows[b], gsems[b]).wait()

        def out_start(c, b):
            pltpu.make_async_copy(
                rows[b],
                emb_hbm.at[pl.ds(base + c * jnp.int32(CHUNK), CHUNK)],
                osems[b]).start()

        def out_wait(c, b):
            pltpu.make_async_copy(
                rows[b],
                emb_hbm.at[pl.ds(base + c * jnp.int32(CHUNK), CHUNK)],
                osems[b]).wait()

        assert n_chunks > NB + LEAD and NB >= 2 * LEAD

        def step_static(c):
            ci = jnp.int32(c)
            gather_wait(ci, c % NB)
            out_start(ci, c % NB)
            if c >= LEAD:
                out_wait(jnp.int32(c - LEAD), (c - LEAD) % NB)
            if c + LEAD < n_chunks:
                gather_start(jnp.int32(c + LEAD), (c + LEAD) % NB)

        n_main = ((n_chunks - LEAD) - NB) // NB
        tail0 = NB + n_main * NB

        for b in range(LEAD):
            gather_start(jnp.int32(b), b)
        for c in range(NB):
            step_static(c)

        @pl.loop(jnp.int32(1), jnp.int32(1 + n_main))
        def gather_body(g):
            for b in range(NB):
                c = g * jnp.int32(NB) + jnp.int32(b)
                gather_wait(c, b)
                out_start(c, b)
                out_wait(c - jnp.int32(LEAD), (b - LEAD) % NB)
                gather_start(c + jnp.int32(LEAD), (b + LEAD) % NB)

        for c in range(tail0, n_chunks):
            step_static(c)
        for c in range(n_chunks - LEAD, n_chunks):
            out_wait(jnp.int32(c), c % NB)

    return k(ids32, p1, p2, table)


def _mm_body(a_ref, w_ref, o_ref):
    o_ref[...] = jnp.dot(a_ref[...], w_ref[...],
                         preferred_element_type=jnp.float32)


def _mm_body_acc(a_ref, w_ref, oin_ref, o_ref):
    del oin_ref
    o_ref[...] = jnp.dot(a_ref[...], w_ref[...],
                         preferred_element_type=jnp.float32)


TM = 10240
SLICES = 1


def _tc_matmul_slice(emb, wt, out_buf, block0):
    nb = emb.shape[0] // TM
    if out_buf is None:
        return pl.pallas_call(
            _mm_body,
            grid=(nb,),
            in_specs=[
                pl.BlockSpec((TM, EMBED_DIM), lambda i: (i, i - i)),
                pl.BlockSpec((EMBED_DIM, MODEL_DIM),
                             lambda i: (i - i, i - i)),
            ],
            out_specs=pl.BlockSpec(
                (TM, MODEL_DIM), lambda i: (i + block0, i - i)),
            out_shape=jax.ShapeDtypeStruct((N_TOK, MODEL_DIM), jnp.float32),
        )(emb, wt)
    return pl.pallas_call(
        _mm_body_acc,
        grid=(nb,),
        in_specs=[
            pl.BlockSpec((TM, EMBED_DIM), lambda i: (i, i - i)),
            pl.BlockSpec((EMBED_DIM, MODEL_DIM), lambda i: (i - i, i - i)),
            pl.BlockSpec((TM, MODEL_DIM), lambda i: (i + block0, i - i)),
        ],
        out_specs=pl.BlockSpec(
            (TM, MODEL_DIM), lambda i: (i + block0, i - i)),
        out_shape=jax.ShapeDtypeStruct((N_TOK, MODEL_DIM), jnp.float32),
        input_output_aliases={2: 0},
    )(emb, wt, out_buf)


def kernel(ids, table, W):
    ids32 = ids.astype(jnp.int32)
    p1 = jnp.concatenate([ids32[:, :1], ids32[:, :-1]], axis=1)
    p2 = jnp.concatenate([ids32[:, :2], ids32[:, :-2]], axis=1)
    ids_f = ids32.reshape(-1)
    p1_f = p1.reshape(-1)
    p2_f = p2.reshape(-1)
    wt = W.T

    S = N_TOK // SLICES
    embs = [
        _sc_hash_gather(ids_f[k * S:(k + 1) * S], p1_f[k * S:(k + 1) * S],
                        p2_f[k * S:(k + 1) * S], table)
        for k in range(SLICES)
    ]
    out = None
    for k in range(SLICES):
        out = _tc_matmul_slice(embs[k], wt, out, k * (S // TM))
    return out.reshape(B, L, MODEL_DIM)


# speedup vs baseline: 1.4409x; 1.0013x over previous
import functools

import jax
import jax.numpy as jnp
from jax import lax
from jax.experimental import pallas as pl
from jax.experimental.pallas import tpu as pltpu
from jax.experimental.pallas import tpu_sc as plsc

B = 1024
L = 200
NUM_BUCKETS = 262144
EMBED_DIM = 128
MODEL_DIM = 512
N_TOK = B * L

C1 = 1000003 % NUM_BUCKETS
C2 = (1000003 * 1000003) % NUM_BUCKETS
MASK = NUM_BUCKETS - 1

CHUNK = 128
_NB = 6
_LEAD = 3


def _sc_hash_gather(ids32, p1, p2, table):
    n_tok = ids32.shape[0]
    info = plsc.get_sparse_core_info()
    nw = info.num_cores * info.num_subcores
    tpw = n_tok // nw
    n_chunks = tpw // CHUNK
    mesh = plsc.VectorSubcoreMesh(core_axis_name="c", subcore_axis_name="s")

    @functools.partial(
        pl.kernel,
        mesh=mesh,
        out_type=jax.ShapeDtypeStruct((n_tok, EMBED_DIM), jnp.float32),
        scratch_types=[
            pltpu.VMEM((tpw,), jnp.int32),
            pltpu.VMEM((tpw,), jnp.int32),
            pltpu.VMEM((tpw,), jnp.int32),
            pltpu.VMEM((n_chunks, CHUNK), jnp.int32),
        ] + [pltpu.VMEM((CHUNK, EMBED_DIM), jnp.float32)] * _NB
          + [pltpu.SemaphoreType.DMA] * (2 * _NB),
    )
    def k(ids_hbm, p1_hbm, p2_hbm, table_hbm, emb_hbm,
          ids_v, p1_v, p2_v, idx_v, *bufs):
        wid = (lax.axis_index("s").astype(jnp.int32) * jnp.int32(info.num_cores)
               + lax.axis_index("c").astype(jnp.int32))
        base = wid * jnp.int32(tpw)
        pltpu.sync_copy(ids_hbm.at[pl.ds(base, tpw)], ids_v)
        pltpu.sync_copy(p1_hbm.at[pl.ds(base, tpw)], p1_v)
        pltpu.sync_copy(p2_hbm.at[pl.ds(base, tpw)], p2_v)

        c1 = jnp.int32(C1)
        c2 = jnp.int32(C2)
        mask = jnp.int32(MASK)

        def hash_chunk(c):
            for j in range(CHUNK // 16):
                off = c * jnp.int32(CHUNK) + jnp.int32(j * 16)
                h = (p2_v[pl.ds(off, 16)] * c2
                     + p1_v[pl.ds(off, 16)] * c1
                     + ids_v[pl.ds(off, 16)]) & mask
                idx_v[c, pl.ds(j * 16, 16)] = h

        rows = bufs[:_NB]
        gsems = bufs[_NB:2 * _NB]
        osems = bufs[2 * _NB:]
        NB = _NB
        LEAD = _LEAD

        def gather_start(c, b):
            pltpu.make_async_copy(
                table_hbm.at[idx_v.at[c]], rows[b], gsems[b]).start()

        def gather_wait(c, b):
            pltpu.make_async_copy(
                table_hbm.at[idx_v.at[c]], rows[b], gsems[b]).wait()

        def out_start(c, b):
            pltpu.make_async_copy(
                rows[b],
                emb_hbm.at[pl.ds(base + c * jnp.int32(CHUNK), CHUNK)],
                osems[b]).start()

        def out_wait(c, b):
            pltpu.make_async_copy(
                rows[b],
                emb_hbm.at[pl.ds(base + c * jnp.int32(CHUNK), CHUNK)],
                osems[b]).wait()

        assert n_chunks > NB + LEAD and NB >= 2 * LEAD

        def step_static(c):
            ci = jnp.int32(c)
            gather_wait(ci, c % NB)
            out_start(ci, c % NB)
            if c >= LEAD:
                out_wait(jnp.int32(c - LEAD), (c - LEAD) % NB)
            if c + LEAD < n_chunks:
                hash_chunk(jnp.int32(c + LEAD))
                gather_start(jnp.int32(c + LEAD), (c + LEAD) % NB)

        n_main = ((n_chunks - LEAD) - NB) // NB
        tail0 = NB + n_main * NB

        for b in range(LEAD):
            hash_chunk(jnp.int32(b))
            gather_start(jnp.int32(b), b)
        for c in range(NB):
            step_static(c)

        @pl.loop(jnp.int32(1), jnp.int32(1 + n_main))
        def gather_body(g):
            for b in range(NB):
                c = g * jnp.int32(NB) + jnp.int32(b)
                gather_wait(c, b)
                out_start(c, b)
                out_wait(c - jnp.int32(LEAD), (b - LEAD) % NB)
                hash_chunk(c + jnp.int32(LEAD))
                gather_start(c + jnp.int32(LEAD), (b + LEAD) % NB)

        for c in range(tail0, n_chunks):
            step_static(c)
        for c in range(n_chunks - LEAD, n_chunks):
            out_wait(jnp.int32(c), c % NB)

    return k(ids32, p1, p2, table)


def _mm_body(a_ref, w_ref, o_ref):
    o_ref[...] = jnp.dot(a_ref[...], w_ref[...],
                         preferred_element_type=jnp.float32)


def _mm_body_acc(a_ref, w_ref, oin_ref, o_ref):
    del oin_ref
    o_ref[...] = jnp.dot(a_ref[...], w_ref[...],
                         preferred_element_type=jnp.float32)


TM = 10240
SLICES = 1


def _tc_matmul_slice(emb, wt, out_buf, block0):
    nb = emb.shape[0] // TM
    if out_buf is None:
        return pl.pallas_call(
            _mm_body,
            grid=(nb,),
            in_specs=[
                pl.BlockSpec((TM, EMBED_DIM), lambda i: (i, i - i)),
                pl.BlockSpec((EMBED_DIM, MODEL_DIM),
                             lambda i: (i - i, i - i)),
            ],
            out_specs=pl.BlockSpec(
                (TM, MODEL_DIM), lambda i: (i + block0, i - i)),
            out_shape=jax.ShapeDtypeStruct((N_TOK, MODEL_DIM), jnp.float32),
        )(emb, wt)
    return pl.pallas_call(
        _mm_body_acc,
        grid=(nb,),
        in_specs=[
            pl.BlockSpec((TM, EMBED_DIM), lambda i: (i, i - i)),
            pl.BlockSpec((EMBED_DIM, MODEL_DIM), lambda i: (i - i, i - i)),
            pl.BlockSpec((TM, MODEL_DIM), lambda i: (i + block0, i - i)),
        ],
        out_specs=pl.BlockSpec(
            (TM, MODEL_DIM), lambda i: (i + block0, i - i)),
        out_shape=jax.ShapeDtypeStruct((N_TOK, MODEL_DIM), jnp.float32),
        input_output_aliases={2: 0},
    )(emb, wt, out_buf)


def kernel(ids, table, W):
    ids32 = ids.astype(jnp.int32)
    p1 = jnp.concatenate([ids32[:, :1], ids32[:, :-1]], axis=1)
    p2 = jnp.concatenate([ids32[:, :2], ids32[:, :-2]], axis=1)
    ids_f = ids32.reshape(-1)
    p1_f = p1.reshape(-1)
    p2_f = p2.reshape(-1)
    wt = W.T

    S = N_TOK // SLICES
    embs = [
        _sc_hash_gather(ids_f[k * S:(k + 1) * S], p1_f[k * S:(k + 1) * S],
                        p2_f[k * S:(k + 1) * S], table)
        for k in range(SLICES)
    ]
    out = None
    for k in range(SLICES):
        out = _tc_matmul_slice(embs[k], wt, out, k * (S // TM))
    return out.reshape(B, L, MODEL_DIM)
